# baseline (speedup 1.0000x reference)
import functools

import jax
import jax.numpy as jnp
from jax import lax
from jax.experimental import pallas as pl
from jax.experimental.pallas import tpu as pltpu
from jax.experimental.pallas import tpu_sc as plsc

_N, _E, _B = 10000, 320000, 64
_FE, _FX, _FU, _H = 16, 128, 128, 32
_NC, _NS = 2, 16
_NW = _NC * _NS
_EPW = _E // _NW
_ECH = 400
_NCH = _EPW // _ECH
_EG = _ECH // 16
_NPW = 320

_f32 = jnp.float32
_i32 = jnp.int32
_HIGH = lax.Precision.HIGHEST


def _mm_exact(a, b):
    k_dim = a.shape[1]
    acc = a[:, 0:1] * b[0:1, :]
    for k in range(1, k_dim):
        acc = acc + a[:, k:k + 1] * b[k:k + 1, :]
    return acc


def _prep_body(u_ref, wqe_ref, bqe_ref, wket_ref, bkec_ref, wqx_ref, bqx_ref,
               wkxt_ref, bkxc_ref, te_ref, ce_ref, tx_ref, cx_ref):
    u = u_ref[...]
    qe = _mm_exact(u, wqe_ref[...]) + bqe_ref[...]
    te_ref[...] = _mm_exact(qe, wket_ref[...])
    ce_ref[...] = _mm_exact(qe, bkec_ref[...])
    qx = _mm_exact(u, wqx_ref[...]) + bqx_ref[...]
    tx_ref[...] = _mm_exact(qx, wkxt_ref[...])
    cx_ref[...] = _mm_exact(qx, bkxc_ref[...])


def _final_body(pe_ref, px_ref, u_ref, wu_ref, bu_ref, out_ref):
    e_agg = pe_ref[0] + pe_ref[1]
    x_agg = px_ref[0] + px_ref[1]
    out = lax.dot_general(x_agg, wu_ref[0:_FX, :], (((1,), (0,)), ((), ())))
    out += lax.dot_general(e_agg, wu_ref[_FX:_FX + _FE, :],
                           (((1,), (0,)), ((), ())))
    out += lax.dot_general(u_ref[...], wu_ref[_FX + _FE:, :],
                           (((1,), (0,)), ((), ())))
    out_ref[...] = out + bu_ref[...]


def _sigmoid(z):
    return 1.0 / (1.0 + jnp.exp(-z))


def _sc_body(ei_hbm, batch_hbm, e_hbm, x_hbm, te_hbm, ce_hbm, tx_hbm, cx_hbm,
             oute_hbm, outx_hbm,
             batch_v, te_v, ce_v, tx_v, cx_v, acce_v, accx_v, idx_v,
             srcb_v, eb_v, x_v, she_sh, shx_sh,
             sem_s0, sem_s1, sem_e0, sem_e1, sem_x):
    cid = lax.axis_index("c")
    sid = lax.axis_index("s")
    wid = sid * _NC + cid
    iota = lax.iota(_i32, 16)
    ebase = wid * _EPW
    nbase = wid * _NPW
    sem_s = (sem_s0, sem_s1)
    sem_e = (sem_e0, sem_e1)

    pltpu.async_copy(x_hbm.at[pl.ds(nbase * _FX, 80 * _FX)],
                     x_v.at[pl.ds(0, 80 * _FX)], sem_x)

    @pl.when(wid < _NW - 1)
    def _stage_rest():
        pltpu.async_copy(x_hbm.at[pl.ds((nbase + 80) * _FX,
                                        (_NPW - 80) * _FX)],
                         x_v.at[pl.ds(80 * _FX, (_NPW - 80) * _FX)], sem_x)

    def _issue_edge(ch, buf):
        base = ebase + ch * _ECH
        pltpu.async_copy(ei_hbm.at[0, pl.ds(base, _ECH)], srcb_v.at[buf],
                         sem_s[buf])
        pltpu.async_copy(e_hbm.at[pl.ds(base * _FE, _ECH * _FE)],
                         eb_v.at[buf], sem_e[buf])

    def _wait_edge(buf):
        pltpu.make_async_copy(ei_hbm.at[0, pl.ds(0, _ECH)], srcb_v.at[buf],
                              sem_s[buf]).wait()
        pltpu.make_async_copy(e_hbm.at[pl.ds(0, _ECH * _FE)], eb_v.at[buf],
                              sem_e[buf]).wait()

    _issue_edge(0, 0)
    _issue_edge(1, 1)

    pltpu.sync_copy(batch_hbm, batch_v)
    pltpu.sync_copy(te_hbm, te_v)
    pltpu.sync_copy(ce_hbm, ce_v)
    pltpu.sync_copy(tx_hbm, tx_v)
    pltpu.sync_copy(cx_hbm, cx_v)

    zero16 = jnp.zeros((16,), _f32)

    def _zero_e(r, _):
        acce_v[r, :] = zero16
        return 0

    lax.fori_loop(0, _B, _zero_e, 0)

    def _zero_x(i, _):
        accx_v[i // 8, pl.ds((i % 8) * 16, 16)] = zero16
        return 0

    lax.fori_loop(0, _B * 8, _zero_x, 0)

    for k in range(4):
        idx_v[pl.ds(k * 16, 16)] = iota + k * 16

    @pl.when(sid == 0)
    def _zero_shared():
        pltpu.sync_copy(acce_v, she_sh)
        pltpu.sync_copy(accx_v, shx_sh)

    plsc.subcore_barrier()

    _rotcols = [((jnp.arange(16, dtype=_i32) + f) & 15) for f in range(16)]

    def _process_chunk(buf):
        @plsc.parallel_loop(0, _EG, unroll=5)
        def _grp(g):
            rowbase = g * (16 * _FE) + iota * _FE
            src16 = srcb_v[buf, pl.ds(g * 16, 16)]
            b_v = plsc.load_gather(batch_v, [src16])
            accs = [plsc.load_gather(ce_v, [b_v]), None, None, None]
            efs = []
            for f in range(_FE):
                fv = _rotcols[f]
                e_f = plsc.load_gather(eb_v.at[buf], [rowbase + fv])
                t_f = plsc.load_gather(te_v, [b_v, fv])
                prod = e_f * t_f
                c = f % 4
                accs[c] = prod if accs[c] is None else accs[c] + prod
                efs.append(e_f)
            z = (accs[0] + accs[1]) + (accs[2] + accs[3])
            a = _sigmoid(z)
            for f in range(_FE):
                plsc.addupdate_scatter(acce_v, [b_v, _rotcols[f]],
                                       a * efs[f])

    def _ring(j, _):
        _wait_edge(0)
        _process_chunk(0)

        @pl.when(2 * j + 2 < _NCH)
        def _issue0():
            _issue_edge(2 * j + 2, 0)

        _wait_edge(1)
        _process_chunk(1)

        @pl.when(2 * j + 3 < _NCH)
        def _issue1():
            _issue_edge(2 * j + 3, 1)

        return 0

    with jax.named_scope("edge_phase"):
        lax.fori_loop(0, (_NCH - 1) // 2, _ring, 0)
        _wait_edge(0)
        _process_chunk(0)

    n_groups = jnp.where(wid == _NW - 1, (_N - (_NW - 1) * _NPW) // 16,
                         _NPW // 16)
    pltpu.make_async_copy(x_hbm.at[pl.ds(0, 80 * _FX)],
                          x_v.at[pl.ds(0, 80 * _FX)], sem_x).wait()

    @pl.when(wid < _NW - 1)
    def _wait_rest():
        pltpu.make_async_copy(x_hbm.at[pl.ds(0, (_NPW - 80) * _FX)],
                              x_v.at[pl.ds(80 * _FX, (_NPW - 80) * _FX)],
                              sem_x).wait()

    _rotcols_x = [(f & ~15) + ((jnp.arange(16, dtype=_i32) + f) & 15)
                  for f in range(_FX)]

    def _node_group(g, _):
        rowbase = g * (16 * _FX) + iota * _FX
        b_v = batch_v[pl.ds(nbase + g * 16, 16)]
        accs = [plsc.load_gather(cx_v, [b_v])] + [None] * 7
        for f in range(_FX):
            fv = _rotcols_x[f]
            x_f = plsc.load_gather(x_v, [rowbase + fv])
            t_f = plsc.load_gather(tx_v, [b_v, fv])
            prod = x_f * t_f
            c = f % 8
            accs[c] = prod if accs[c] is None else accs[c] + prod
        z = ((accs[0] + accs[1]) + (accs[2] + accs[3])) + (
            (accs[4] + accs[5]) + (accs[6] + accs[7]))
        a = _sigmoid(z)
        for f in range(_FX):
            x_f = plsc.load_gather(x_v, [rowbase + _rotcols_x[f]])
            plsc.addupdate_scatter(accx_v, [b_v, _rotcols_x[f]], a * x_f)
        return 0

    with jax.named_scope("node_phase"):
        lax.fori_loop(0, n_groups, _node_group, 0)

    with jax.named_scope("reduce_phase"):
        pltpu.sync_copy(acce_v, she_sh.at[idx_v], add=True)
        pltpu.sync_copy(accx_v, shx_sh.at[idx_v], add=True)
        plsc.subcore_barrier()

    @pl.when(sid == 0)
    def _writeback():
        pltpu.sync_copy(she_sh, oute_hbm.at[cid])
        pltpu.sync_copy(shx_sh, outx_hbm.at[cid])


def _make_sc_agg():
    return functools.partial(
        pl.kernel,
        out_type=[jax.ShapeDtypeStruct((_NC, _B, _FE), _f32),
                  jax.ShapeDtypeStruct((_NC, _B, _FX), _f32)],
        mesh=plsc.VectorSubcoreMesh(core_axis_name="c", subcore_axis_name="s",
                                    num_cores=_NC, num_subcores=_NS),
        compiler_params=pltpu.CompilerParams(needs_layout_passes=False,
                                             use_tc_tiling_on_sc=False),
        scratch_types=[
        pltpu.VMEM((_N,), _i32),
        pltpu.VMEM((_B, _FE), _f32),
        pltpu.VMEM((_B,), _f32),
        pltpu.VMEM((_B, _FX), _f32),
        pltpu.VMEM((_B,), _f32),
        pltpu.VMEM((_B, _FE), _f32),
        pltpu.VMEM((_B, _FX), _f32),
        pltpu.VMEM((_B,), _i32),
        pltpu.VMEM((2, _ECH), _i32),
        pltpu.VMEM((2, _ECH * _FE), _f32),
        pltpu.VMEM((_NPW * _FX,), _f32),
            pltpu.VMEM_SHARED((_B, _FE), _f32),
            pltpu.VMEM_SHARED((_B, _FX), _f32),
            pltpu.SemaphoreType.DMA,
            pltpu.SemaphoreType.DMA,
            pltpu.SemaphoreType.DMA,
            pltpu.SemaphoreType.DMA,
            pltpu.SemaphoreType.DMA,
        ],
    )(_sc_body)


def kernel(x, edge_index, e, u, batch, W_u, b_u, W_ke, b_ke, W_qe, b_qe,
           W_kx, b_kx, W_qx, b_qx):
    ei32 = edge_index.astype(_i32)
    batch32 = batch.astype(_i32)

    te, ce, tx, cx = pl.pallas_call(
        _prep_body,
        out_shape=[jax.ShapeDtypeStruct((_B, _FE), _f32),
                   jax.ShapeDtypeStruct((_B, 1), _f32),
                   jax.ShapeDtypeStruct((_B, _FX), _f32),
                   jax.ShapeDtypeStruct((_B, 1), _f32)],
    )(u, W_qe, b_qe.reshape(1, _H), W_ke.T, b_ke.reshape(_H, 1),
      W_qx, b_qx.reshape(1, _H), W_kx.T, b_kx.reshape(_H, 1))

    pe, px = _make_sc_agg()(ei32, batch32, e.reshape(_E * _FE),
                            x.reshape(_N * _FX), te, ce.reshape(_B), tx,
                            cx.reshape(_B))

    out = pl.pallas_call(
        _final_body,
        out_shape=jax.ShapeDtypeStruct((_B, _FU), _f32),
    )(pe, px, u, W_u, b_u.reshape(1, _FU))
    return out


# speedup vs baseline: 1.0298x; 1.0298x over previous
import functools

import jax
import jax.numpy as jnp
from jax import lax
from jax.experimental import pallas as pl
from jax.experimental.pallas import tpu as pltpu
from jax.experimental.pallas import tpu_sc as plsc

_N, _E, _B = 10000, 320000, 64
_FE, _FX, _FU, _H = 16, 128, 128, 32
_NC, _NS = 2, 16
_NW = _NC * _NS
_EPW = _E // _NW
_ECH = 400
_NCH = _EPW // _ECH
_EG = _ECH // 16
_NPW = 320

_f32 = jnp.float32
_i32 = jnp.int32
_HIGH = lax.Precision.HIGHEST


def _mm_exact(a, b):
    k_dim = a.shape[1]
    acc = a[:, 0:1] * b[0:1, :]
    for k in range(1, k_dim):
        acc = acc + a[:, k:k + 1] * b[k:k + 1, :]
    return acc


def _prep_body(u_ref, wqe_ref, bqe_ref, wket_ref, bkec_ref, wqx_ref, bqx_ref,
               wkxt_ref, bkxc_ref, te_ref, ce_ref, tx_ref, cx_ref):
    u = u_ref[...]
    qe = _mm_exact(u, wqe_ref[...]) + bqe_ref[...]
    te_ref[...] = _mm_exact(qe, wket_ref[...])
    ce_ref[...] = _mm_exact(qe, bkec_ref[...])
    qx = _mm_exact(u, wqx_ref[...]) + bqx_ref[...]
    tx_ref[...] = _mm_exact(qx, wkxt_ref[...])
    cx_ref[...] = _mm_exact(qx, bkxc_ref[...])


def _final_body(pe_ref, px_ref, u_ref, wu_ref, bu_ref, out_ref):
    e_agg = pe_ref[0] + pe_ref[1]
    x_agg = px_ref[0] + px_ref[1]
    out = lax.dot_general(x_agg, wu_ref[0:_FX, :], (((1,), (0,)), ((), ())))
    out += lax.dot_general(e_agg, wu_ref[_FX:_FX + _FE, :],
                           (((1,), (0,)), ((), ())))
    out += lax.dot_general(u_ref[...], wu_ref[_FX + _FE:, :],
                           (((1,), (0,)), ((), ())))
    out_ref[...] = out + bu_ref[...]


def _sigmoid(z):
    return 1.0 / (1.0 + jnp.exp(-z))


def _sc_body(ei_hbm, batch_hbm, e_hbm, x_hbm, te_hbm, ce_hbm, tx_hbm, cx_hbm,
             oute_hbm, outx_hbm,
             batch_v, te_v, ce_v, tx_v, cx_v, acce_v, accx_v, idx_v,
             srcb_v, eb_v, x_v, she_sh, shx_sh,
             sem_s0, sem_s1, sem_e0, sem_e1, sem_x):
    cid = lax.axis_index("c")
    sid = lax.axis_index("s")
    wid = sid * _NC + cid
    iota = lax.iota(_i32, 16)
    ebase = wid * _EPW
    nbase = wid * _NPW
    sem_s = (sem_s0, sem_s1)
    sem_e = (sem_e0, sem_e1)

    pltpu.async_copy(x_hbm.at[pl.ds(nbase, 80)], x_v.at[pl.ds(0, 80)], sem_x)

    @pl.when(wid < _NW - 1)
    def _stage_rest():
        pltpu.async_copy(x_hbm.at[pl.ds(nbase + 80, _NPW - 80)],
                         x_v.at[pl.ds(80, _NPW - 80)], sem_x)

    def _issue_edge(ch, buf):
        base = ebase + ch * _ECH
        pltpu.async_copy(ei_hbm.at[0, pl.ds(base, _ECH)], srcb_v.at[buf],
                         sem_s[buf])
        pltpu.async_copy(e_hbm.at[pl.ds(base, _ECH)], eb_v.at[buf],
                         sem_e[buf])

    def _wait_edge(buf):
        pltpu.make_async_copy(ei_hbm.at[0, pl.ds(0, _ECH)], srcb_v.at[buf],
                              sem_s[buf]).wait()
        pltpu.make_async_copy(e_hbm.at[pl.ds(0, _ECH)], eb_v.at[buf],
                              sem_e[buf]).wait()

    _issue_edge(0, 0)
    _issue_edge(1, 1)

    pltpu.sync_copy(batch_hbm, batch_v)
    pltpu.sync_copy(te_hbm, te_v)
    pltpu.sync_copy(ce_hbm, ce_v)
    pltpu.sync_copy(tx_hbm, tx_v)
    pltpu.sync_copy(cx_hbm, cx_v)

    zero16 = jnp.zeros((16,), _f32)

    def _zero_e(r, _):
        acce_v[r, :] = zero16
        return 0

    lax.fori_loop(0, _B, _zero_e, 0)

    def _zero_x(i, _):
        accx_v[i // 8, pl.ds((i % 8) * 16, 16)] = zero16
        return 0

    lax.fori_loop(0, _B * 8, _zero_x, 0)

    for k in range(4):
        idx_v[pl.ds(k * 16, 16)] = iota + k * 16

    @pl.when(sid == 0)
    def _zero_shared():
        pltpu.sync_copy(acce_v, she_sh)
        pltpu.sync_copy(accx_v, shx_sh)

    plsc.subcore_barrier()

    _rotcols = [((jnp.arange(16, dtype=_i32) + f) & 15) for f in range(16)]

    def _process_chunk(buf):
        @plsc.parallel_loop(0, _EG, unroll=5)
        def _grp(g):
            rows = g * 16 + iota
            src16 = srcb_v[buf, pl.ds(g * 16, 16)]
            b_v = plsc.load_gather(batch_v, [src16])
            accs = [plsc.load_gather(ce_v, [b_v]), None, None, None]
            efs = []
            for f in range(_FE):
                fv = _rotcols[f]
                e_f = plsc.load_gather(eb_v.at[buf], [rows, fv])
                t_f = plsc.load_gather(te_v, [b_v, fv])
                prod = e_f * t_f
                c = f % 4
                accs[c] = prod if accs[c] is None else accs[c] + prod
                efs.append(e_f)
            z = (accs[0] + accs[1]) + (accs[2] + accs[3])
            a = _sigmoid(z)
            for f in range(_FE):
                plsc.addupdate_scatter(acce_v, [b_v, _rotcols[f]],
                                       a * efs[f])

    def _ring(j, _):
        _wait_edge(0)
        _process_chunk(0)

        @pl.when(2 * j + 2 < _NCH)
        def _issue0():
            _issue_edge(2 * j + 2, 0)

        _wait_edge(1)
        _process_chunk(1)

        @pl.when(2 * j + 3 < _NCH)
        def _issue1():
            _issue_edge(2 * j + 3, 1)

        return 0

    with jax.named_scope("edge_phase"):
        lax.fori_loop(0, (_NCH - 1) // 2, _ring, 0)
        _wait_edge(0)
        _process_chunk(0)

    n_groups = jnp.where(wid == _NW - 1, (_N - (_NW - 1) * _NPW) // 16,
                         _NPW // 16)
    pltpu.make_async_copy(x_hbm.at[pl.ds(0, 80)], x_v.at[pl.ds(0, 80)],
                          sem_x).wait()

    @pl.when(wid < _NW - 1)
    def _wait_rest():
        pltpu.make_async_copy(x_hbm.at[pl.ds(0, _NPW - 80)],
                              x_v.at[pl.ds(80, _NPW - 80)], sem_x).wait()

    _rotcols_x = [(f & ~15) + ((jnp.arange(16, dtype=_i32) + f) & 15)
                  for f in range(_FX)]

    def _node_group(g, _):
        rows = g * 16 + iota
        b_v = batch_v[pl.ds(nbase + g * 16, 16)]
        accs = [plsc.load_gather(cx_v, [b_v])] + [None] * 7
        for f in range(_FX):
            fv = _rotcols_x[f]
            x_f = plsc.load_gather(x_v, [rows, fv])
            t_f = plsc.load_gather(tx_v, [b_v, fv])
            prod = x_f * t_f
            c = f % 8
            accs[c] = prod if accs[c] is None else accs[c] + prod
        z = ((accs[0] + accs[1]) + (accs[2] + accs[3])) + (
            (accs[4] + accs[5]) + (accs[6] + accs[7]))
        a = _sigmoid(z)
        for f in range(_FX):
            x_f = plsc.load_gather(x_v, [rows, _rotcols_x[f]])
            plsc.addupdate_scatter(accx_v, [b_v, _rotcols_x[f]], a * x_f)
        return 0

    with jax.named_scope("node_phase"):
        lax.fori_loop(0, n_groups, _node_group, 0)

    with jax.named_scope("reduce_phase"):
        pltpu.sync_copy(acce_v, she_sh.at[idx_v], add=True)
        pltpu.sync_copy(accx_v, shx_sh.at[idx_v], add=True)
        plsc.subcore_barrier()

    @pl.when(sid == 0)
    def _writeback():
        pltpu.sync_copy(she_sh, oute_hbm.at[cid])
        pltpu.sync_copy(shx_sh, outx_hbm.at[cid])


def _make_sc_agg():
    return functools.partial(
        pl.kernel,
        out_type=[jax.ShapeDtypeStruct((_NC, _B, _FE), _f32),
                  jax.ShapeDtypeStruct((_NC, _B, _FX), _f32)],
        mesh=plsc.VectorSubcoreMesh(core_axis_name="c", subcore_axis_name="s",
                                    num_cores=_NC, num_subcores=_NS),
        compiler_params=pltpu.CompilerParams(needs_layout_passes=False,
                                             use_tc_tiling_on_sc=False),
        scratch_types=[
        pltpu.VMEM((_N,), _i32),
        pltpu.VMEM((_B, _FE), _f32),
        pltpu.VMEM((_B,), _f32),
        pltpu.VMEM((_B, _FX), _f32),
        pltpu.VMEM((_B,), _f32),
        pltpu.VMEM((_B, _FE), _f32),
        pltpu.VMEM((_B, _FX), _f32),
        pltpu.VMEM((_B,), _i32),
        pltpu.VMEM((2, _ECH), _i32),
        pltpu.VMEM((2, _ECH, _FE), _f32),
        pltpu.VMEM((_NPW, _FX), _f32),
            pltpu.VMEM_SHARED((_B, _FE), _f32),
            pltpu.VMEM_SHARED((_B, _FX), _f32),
            pltpu.SemaphoreType.DMA,
            pltpu.SemaphoreType.DMA,
            pltpu.SemaphoreType.DMA,
            pltpu.SemaphoreType.DMA,
            pltpu.SemaphoreType.DMA,
        ],
    )(_sc_body)


def kernel(x, edge_index, e, u, batch, W_u, b_u, W_ke, b_ke, W_qe, b_qe,
           W_kx, b_kx, W_qx, b_qx):
    ei32 = edge_index.astype(_i32)
    batch32 = batch.astype(_i32)

    te, ce, tx, cx = pl.pallas_call(
        _prep_body,
        out_shape=[jax.ShapeDtypeStruct((_B, _FE), _f32),
                   jax.ShapeDtypeStruct((_B, 1), _f32),
                   jax.ShapeDtypeStruct((_B, _FX), _f32),
                   jax.ShapeDtypeStruct((_B, 1), _f32)],
    )(u, W_qe, b_qe.reshape(1, _H), W_ke.T, b_ke.reshape(_H, 1),
      W_qx, b_qx.reshape(1, _H), W_kx.T, b_kx.reshape(_H, 1))

    pe, px = _make_sc_agg()(ei32, batch32, e, x, te, ce.reshape(_B), tx,
                            cx.reshape(_B))

    out = pl.pallas_call(
        _final_body,
        out_shape=jax.ShapeDtypeStruct((_B, _FU), _f32),
    )(pe, px, u, W_u, b_u.reshape(1, _FU))
    return out
